# baseline (speedup 1.0000x reference)
import functools

import jax
import jax.numpy as jnp
from jax import lax
from jax.experimental import pallas as pl
from jax.experimental.pallas import tpu as pltpu
from jax.experimental.pallas import tpu_sc as plsc

_NSAMPLE = 32


def _ceil_to(x, m):
    return (x + m - 1) // m * m


def _pad_last(x, to):
    pad = to - x.shape[-1]
    if pad == 0:
        return x
    return jnp.pad(x, [(0, 0)] * (x.ndim - 1) + [(0, pad)])


def _pad_rows(w, to):
    pad = to - w.shape[0]
    if pad == 0:
        return w
    return jnp.pad(w, [(0, pad), (0, 0)])


def _knn_select(d, iota, nsel):
    R, N = d.shape
    inf = jnp.float32(jnp.inf)
    sel = lax.broadcasted_iota(jnp.int32, (R, nsel), 1)
    if R * N <= 64 * 256:
        unroll = nsel
    else:
        unroll = 4 if nsel % 4 == 0 else nsel

    def step(s, carry):
        m, am, out_v, out_i = carry
        for k in range(unroll):
            keep = (d > m) | ((d == m) & (iota > am))
            cand = jnp.where(keep, d, inf)
            m = jnp.min(cand, axis=1, keepdims=True)
            am = jnp.min(jnp.where(cand == m, iota, N), axis=1, keepdims=True)
            j = s * unroll + k
            out_v = jnp.where(sel == j, m, out_v)
            out_i = jnp.where(sel == j, am, out_i)
        return m, am, out_v, out_i

    init = (jnp.full((R, 1), -inf, jnp.float32),
            jnp.full((R, 1), -1, jnp.int32),
            jnp.zeros((R, nsel), jnp.float32),
            jnp.zeros((R, nsel), jnp.int32))
    _, am_, out_v, out_i = lax.fori_loop(0, nsel // unroll, step, init)
    return out_v, out_i


def _sq_dist(a, b):
    asq = jnp.sum(a * a, axis=1, keepdims=True)
    bsq = jnp.sum(b * b, axis=1)
    prod = lax.dot_general(a, b, (((1,), (1,)), ((), ())),
                           preferred_element_type=jnp.float32)
    return asq - 2.0 * prod + bsq[None, :]


def _lex_select(v, key, nsel, unroll):
    R, N = v.shape
    inf = jnp.float32(jnp.inf)
    big = jnp.int32(2**31 - 1)
    sel = lax.broadcasted_iota(jnp.int32, (R, nsel), 1)

    def step(s, carry):
        m, am, out_v, out_i = carry
        for k in range(unroll):
            keep = (v > m) | ((v == m) & (key > am))
            cand = jnp.where(keep, v, inf)
            m = jnp.min(cand, axis=1, keepdims=True)
            am = jnp.min(jnp.where(cand == m, key, big), axis=1, keepdims=True)
            j = s * unroll + k
            out_v = jnp.where(sel == j, m, out_v)
            out_i = jnp.where(sel == j, am, out_i)
        return m, am, out_v, out_i

    init = (jnp.full((R, 1), -inf, jnp.float32),
            jnp.full((R, 1), -1, jnp.int32),
            jnp.zeros((R, nsel), jnp.float32),
            jnp.zeros((R, nsel), jnp.int32))
    _, _, out_v, out_i = lax.fori_loop(0, nsel // unroll, step, init)
    return out_v, out_i


def _knn_select_grouped(d, nsel, g):
    R, N = d.shape
    Q = N // g
    D = [d[:, j * Q:(j + 1) * Q] for j in range(g)]
    iq = lax.broadcasted_iota(jnp.int32, (R, Q), 1)
    G = D[0]
    GI = iq
    for j in range(1, g):
        lt = D[j] < G
        G = jnp.where(lt, D[j], G)
        GI = jnp.where(lt, j * Q + iq, GI)

    inf = jnp.float32(jnp.inf)
    big = jnp.int32(2**31 - 1)
    nc = g * nsel
    selc = lax.broadcasted_iota(jnp.int32, (R, nc), 1)

    def step(s, carry):
        m, am, cv, ci = carry
        keep = (G > m) | ((G == m) & (GI > am))
        cand = jnp.where(keep, G, inf)
        m = jnp.min(cand, axis=1, keepdims=True)
        am = jnp.min(jnp.where(cand == m, GI, big), axis=1, keepdims=True)
        p = lax.bitwise_and(am, Q - 1)
        hit = iq == p
        for j in range(g):
            vj = jnp.min(jnp.where(hit, D[j], inf), axis=1, keepdims=True)
            cv = jnp.where(selc == s * g + j, vj, cv)
            ci = jnp.where(selc == s * g + j, j * Q + p, ci)
        return m, am, cv, ci

    init = (jnp.full((R, 1), -inf, jnp.float32),
            jnp.full((R, 1), -1, jnp.int32),
            jnp.zeros((R, nc), jnp.float32),
            jnp.zeros((R, nc), jnp.int32))
    _, _, cv, ci = lax.fori_loop(0, nsel, step, init)
    return _lex_select(cv, ci, nsel, unroll=4 if nsel % 4 == 0 else nsel)


def _topk_body(N, nsel, x_ref, c_ref, o_ref):
    x = x_ref[0]
    c = c_ref[0]
    d = _sq_dist(c, x)
    Rb = d.shape[0]
    if N >= 2048:
        _, idx = _knn_select_grouped(d, nsel, 4)
    else:
        iota = lax.broadcasted_iota(jnp.int32, (Rb, N), 1)
        _, idx = _knn_select(d, iota, nsel)
    b = pl.program_id(0)
    o_ref[0] = idx + b * N


def _topk(xyzp, cent, nsel, rb):
    B, N, _ = xyzp.shape
    npoint = cent.shape[1]
    body = functools.partial(_topk_body, N, nsel)
    return pl.pallas_call(
        body,
        grid=(B, npoint // rb),
        in_specs=[
            pl.BlockSpec((1, N, 16), lambda b, r: (b, 0, 0)),
            pl.BlockSpec((1, rb, 16), lambda b, r: (b, r, 0)),
        ],
        out_specs=pl.BlockSpec((1, rb, nsel), lambda b, r: (b, r, 0)),
        out_shape=jax.ShapeDtypeStruct((B, npoint, nsel), jnp.int32),
    )(xyzp, cent)


def _sc_gather(table, idx, chunk):
    V, Cp = table.shape
    (btot,) = idx.shape
    info = plsc.get_sparse_core_info()
    nc, ns = info.num_cores, info.num_subcores
    nw = nc * ns
    per_w = btot // nw
    assert per_w * nw == btot and per_w % chunk == 0 and chunk % 8 == 0
    nch = per_w // chunk
    mesh = plsc.VectorSubcoreMesh(core_axis_name="c", subcore_axis_name="s")

    @functools.partial(
        pl.kernel, mesh=mesh,
        out_type=jax.ShapeDtypeStruct((btot, Cp), jnp.float32),
        compiler_params=pltpu.CompilerParams(use_tc_tiling_on_sc=False),
        scratch_types=[
            pltpu.VMEM((chunk,), jnp.int32),
            pltpu.VMEM((chunk, Cp), jnp.float32),
            pltpu.SemaphoreType.DMA,
        ],
    )
    def k(table_hbm, idx_hbm, out_hbm, idx_v, rows_v, sem):
        wid = lax.axis_index("s") * nc + lax.axis_index("c")
        for j in range(nch):
            base = wid * per_w + j * chunk
            pltpu.sync_copy(idx_hbm.at[pl.ds(base, chunk)], idx_v)
            pltpu.async_copy(table_hbm.at[idx_v], rows_v, sem).wait()
            pltpu.sync_copy(rows_v, out_hbm.at[pl.ds(base, chunk)])

    return k(table, idx)


def _sa_mlp_body(nlayer, *refs):
    g_ref, c_ref = refs[0], refs[1]
    wrefs = refs[2:2 + 2 * nlayer]
    o_ref = refs[-1]
    g = g_ref[...]
    c = c_ref[...]
    bc, ns, cp = g.shape
    x = (g - c[:, None, :]).reshape(bc * ns, cp)
    for i in range(nlayer):
        w = wrefs[2 * i][...]
        b = wrefs[2 * i + 1][...]
        x = jax.nn.relu(
            lax.dot_general(x, w, (((1,), (0,)), ((), ())),
                            preferred_element_type=jnp.float32) + b)
    cout = x.shape[-1]
    o_ref[...] = jnp.max(x.reshape(bc, ns, cout), axis=1)


def _sa_mlp(g, centp, ws, bc):
    M, ns, cp = g.shape
    cout = ws[-1][0].shape[1]
    nlayer = len(ws)
    args = [g, centp]
    in_specs = [
        pl.BlockSpec((bc, ns, cp), lambda i: (i, 0, 0)),
        pl.BlockSpec((bc, cp), lambda i: (i, 0)),
    ]
    for (w, b) in ws:
        args += [w, b.reshape(1, -1)]
        in_specs += [
            pl.BlockSpec(w.shape, lambda i: (0, 0)),
            pl.BlockSpec((1, b.shape[0]), lambda i: (0, 0)),
        ]
    body = functools.partial(_sa_mlp_body, nlayer)
    return pl.pallas_call(
        body,
        grid=(M // bc,),
        in_specs=in_specs,
        out_specs=pl.BlockSpec((bc, cout), lambda i: (i, 0)),
        out_shape=jax.ShapeDtypeStruct((M, cout), jnp.float32),
    )(*args)


def _fp_body(nlayer, has_fc, N2, *refs):
    i = 0
    x1_ref = refs[i]; i += 1
    x2_ref = refs[i]; i += 1
    f1_ref = refs[i]; i += 1
    f2_ref = refs[i]; i += 1
    w1a_ref = refs[i]; i += 1
    w1b_ref = refs[i]; i += 1
    b1_ref = refs[i]; i += 1
    rest = refs[i:-1]
    o_ref = refs[-1]

    x1 = x1_ref[0]
    x2 = x2_ref[0]
    d = _sq_dist(x1, x2)
    Rb = d.shape[0]
    iota = lax.broadcasted_iota(jnp.int32, (Rb, N2), 1)
    if N2 >= 256:
        vals, idxs = _knn_select_grouped(d, 3, 4)
    else:
        vals, idxs = _knn_select(d, iota, 3)
    dist = jnp.maximum(vals, 1e-10)
    w = 1.0 / dist
    w = w / jnp.sum(w, axis=1, keepdims=True)
    wsp = jnp.zeros_like(d)
    for s in range(3):
        wsp = wsp + jnp.where(iota == idxs[:, s][:, None], w[:, s][:, None], 0.0)
    interp = lax.dot_general(wsp, f2_ref[0], (((1,), (0,)), ((), ())),
                             preferred_element_type=jnp.float32)
    h = (lax.dot_general(interp, w1b_ref[...], (((1,), (0,)), ((), ())),
                         preferred_element_type=jnp.float32)
         + lax.dot_general(f1_ref[0], w1a_ref[...], (((1,), (0,)), ((), ())),
                           preferred_element_type=jnp.float32)
         + b1_ref[...])
    x = jax.nn.relu(h)
    for l in range(nlayer - 1):
        w_ = rest[2 * l][...]
        b_ = rest[2 * l + 1][...]
        x = jax.nn.relu(
            lax.dot_general(x, w_, (((1,), (0,)), ((), ())),
                            preferred_element_type=jnp.float32) + b_)
    if has_fc:
        wf1 = rest[2 * (nlayer - 1)][...]
        bf1 = rest[2 * (nlayer - 1) + 1][...]
        wf2 = rest[2 * (nlayer - 1) + 2][...]
        bf2 = rest[2 * (nlayer - 1) + 3][...]
        x = jax.nn.relu(
            lax.dot_general(x, wf1, (((1,), (0,)), ((), ())),
                            preferred_element_type=jnp.float32) + bf1)
        x = (lax.dot_general(x, wf2, (((1,), (0,)), ((), ())),
                             preferred_element_type=jnp.float32) + bf2)
        x = x[:, :o_ref.shape[-1]]
    o_ref[0] = x


def _fp_stage(x1p, x2p, f1, f2, ws, rb, fc=None):
    B, N1, _ = x1p.shape
    N2 = x2p.shape[1]
    c1p = f1.shape[-1]
    w1, b1 = ws[0]
    c1 = w1.shape[0] - f2.shape[-1]
    w1a = _pad_rows(w1[:c1], c1p)
    w1b = w1[c1:]
    nlayer = len(ws)
    args = [x1p, x2p, f1, f2, w1a, w1b, b1.reshape(1, -1)]
    in_specs = [
        pl.BlockSpec((1, rb, 16), lambda b, r: (b, r, 0)),
        pl.BlockSpec((1, N2, 16), lambda b, r: (b, 0, 0)),
        pl.BlockSpec((1, rb, c1p), lambda b, r: (b, r, 0)),
        pl.BlockSpec((1, N2, f2.shape[-1]), lambda b, r: (b, 0, 0)),
        pl.BlockSpec(w1a.shape, lambda b, r: (0, 0)),
        pl.BlockSpec(w1b.shape, lambda b, r: (0, 0)),
        pl.BlockSpec((1, b1.shape[0]), lambda b, r: (0, 0)),
    ]
    for (w_, b_) in ws[1:]:
        args += [w_, b_.reshape(1, -1)]
        in_specs += [
            pl.BlockSpec(w_.shape, lambda b, r: (0, 0)),
            pl.BlockSpec((1, b_.shape[0]), lambda b, r: (0, 0)),
        ]
    if fc is not None:
        (wf1, bf1), (wf2, bf2) = fc
        wf2p = _pad_last(wf2, 16)
        bf2p = _pad_last(bf2.reshape(1, -1), 16)
        args += [wf1, bf1.reshape(1, -1), wf2p, bf2p]
        in_specs += [
            pl.BlockSpec(wf1.shape, lambda b, r: (0, 0)),
            pl.BlockSpec((1, bf1.shape[0]), lambda b, r: (0, 0)),
            pl.BlockSpec(wf2p.shape, lambda b, r: (0, 0)),
            pl.BlockSpec((1, 16), lambda b, r: (0, 0)),
        ]
        cout = wf2.shape[1]
    else:
        cout = ws[-1][0].shape[1]
    body = functools.partial(_fp_body, nlayer, fc is not None, N2)
    return pl.pallas_call(
        body,
        grid=(B, N1 // rb),
        in_specs=in_specs,
        out_specs=pl.BlockSpec((1, rb, cout), lambda b, r: (b, r, 0)),
        out_shape=jax.ShapeDtypeStruct((B, N1, cout), jnp.float32),
    )(*args)


_SA_CFG = [
    (1024, 1024, 512, 2048),
    (256, 256, 256, 1024),
    (64, 64, 64, 512),
    (16, 16, 16, 128),
]


def _sa_stage(xyzp, feats, ws, npoint, rb, bc, chunk):
    B, N, _ = xyzp.shape
    C = feats.shape[-1]
    cp = _ceil_to(3 + C, 16)
    stride = N // npoint
    cent = xyzp[:, ::stride]
    knn = _topk(xyzp, cent, _NSAMPLE, rb)
    xf = _pad_last(jnp.concatenate([xyzp[..., :3], feats], axis=-1), cp)
    g = _sc_gather(xf.reshape(B * N, cp), knn.reshape(-1), chunk)
    centp = _pad_last(cent[..., :3], cp).reshape(B * npoint, cp)
    wpad = [(_pad_rows(ws[0][0], cp), ws[0][1])] + list(ws[1:])
    nf = _sa_mlp(g.reshape(B * npoint, _NSAMPLE, cp), centp, wpad, bc)
    return cent, nf.reshape(B, npoint, -1)


def kernel(pointcloud, params):
    B, N, _ = pointcloud.shape
    xyzp = _pad_last(pointcloud[..., :3], 16)
    feats0 = pointcloud[..., 3:]

    l_xyz = [xyzp]
    l_f = [feats0]
    for i, (npoint, rb, bc, chunk) in enumerate(_SA_CFG):
        cent, nf = _sa_stage(l_xyz[i], l_f[i], params["sa"][i], npoint, rb, bc, chunk)
        l_xyz.append(cent)
        l_f.append(nf)

    l_f[3] = _fp_stage(l_xyz[3], l_xyz[4], l_f[3], l_f[4], params["fp"][3], rb=64)
    l_f[2] = _fp_stage(l_xyz[2], l_xyz[3], l_f[2], l_f[3], params["fp"][2], rb=256)
    l_f[1] = _fp_stage(l_xyz[1], l_xyz[2], l_f[1], l_f[2], params["fp"][1], rb=1024)
    out = _fp_stage(l_xyz[0], l_xyz[1], l_f[0], l_f[1],
                    params["fp"][0], rb=2048, fc=params["fc"])
    return out


# speedup vs baseline: 1.0517x; 1.0517x over previous
import functools

import jax
import jax.numpy as jnp
from jax import lax
from jax.experimental import pallas as pl
from jax.experimental.pallas import tpu as pltpu
from jax.experimental.pallas import tpu_sc as plsc

_NSAMPLE = 32


def _ceil_to(x, m):
    return (x + m - 1) // m * m


def _pad_last(x, to):
    pad = to - x.shape[-1]
    if pad == 0:
        return x
    return jnp.pad(x, [(0, 0)] * (x.ndim - 1) + [(0, pad)])


def _pad_rows(w, to):
    pad = to - w.shape[0]
    if pad == 0:
        return w
    return jnp.pad(w, [(0, pad), (0, 0)])


def _knn_select(d, iota, nsel):
    R, N = d.shape
    inf = jnp.float32(jnp.inf)
    sel = lax.broadcasted_iota(jnp.int32, (R, nsel), 1)
    if R * N <= 64 * 256:
        unroll = nsel
    else:
        unroll = 4 if nsel % 4 == 0 else nsel

    def step(s, carry):
        m, am, out_v, out_i = carry
        for k in range(unroll):
            keep = (d > m) | ((d == m) & (iota > am))
            cand = jnp.where(keep, d, inf)
            m = jnp.min(cand, axis=1, keepdims=True)
            am = jnp.min(jnp.where(cand == m, iota, N), axis=1, keepdims=True)
            j = s * unroll + k
            out_v = jnp.where(sel == j, m, out_v)
            out_i = jnp.where(sel == j, am, out_i)
        return m, am, out_v, out_i

    init = (jnp.full((R, 1), -inf, jnp.float32),
            jnp.full((R, 1), -1, jnp.int32),
            jnp.zeros((R, nsel), jnp.float32),
            jnp.zeros((R, nsel), jnp.int32))
    _, am_, out_v, out_i = lax.fori_loop(0, nsel // unroll, step, init)
    return out_v, out_i


def _sq_dist(a, b):
    asq = jnp.sum(a * a, axis=1, keepdims=True)
    bsq = jnp.sum(b * b, axis=1)
    prod = lax.dot_general(a, b, (((1,), (1,)), ((), ())),
                           preferred_element_type=jnp.float32)
    return asq - 2.0 * prod + bsq[None, :]


def _lex_select(v, key, nsel, unroll):
    R, N = v.shape
    inf = jnp.float32(jnp.inf)
    big = jnp.int32(2**31 - 1)
    sel = lax.broadcasted_iota(jnp.int32, (R, nsel), 1)

    def step(s, carry):
        m, am, out_v, out_i = carry
        for k in range(unroll):
            keep = (v > m) | ((v == m) & (key > am))
            cand = jnp.where(keep, v, inf)
            m = jnp.min(cand, axis=1, keepdims=True)
            am = jnp.min(jnp.where(cand == m, key, big), axis=1, keepdims=True)
            j = s * unroll + k
            out_v = jnp.where(sel == j, m, out_v)
            out_i = jnp.where(sel == j, am, out_i)
        return m, am, out_v, out_i

    init = (jnp.full((R, 1), -inf, jnp.float32),
            jnp.full((R, 1), -1, jnp.int32),
            jnp.zeros((R, nsel), jnp.float32),
            jnp.zeros((R, nsel), jnp.int32))
    _, _, out_v, out_i = lax.fori_loop(0, nsel // unroll, step, init)
    return out_v, out_i


def _knn_select_grouped(d, nsel, g):
    R, N = d.shape
    Q = N // g
    D = [d[:, j * Q:(j + 1) * Q] for j in range(g)]
    iq = lax.broadcasted_iota(jnp.int32, (R, Q), 1)
    G = D[0]
    GI = iq
    for j in range(1, g):
        lt = D[j] < G
        G = jnp.where(lt, D[j], G)
        GI = jnp.where(lt, j * Q + iq, GI)

    inf = jnp.float32(jnp.inf)
    big = jnp.int32(2**31 - 1)
    nc = g * nsel
    selc = lax.broadcasted_iota(jnp.int32, (R, nc), 1)

    def step(s, carry):
        m, am, cv, ci = carry
        keep = (G > m) | ((G == m) & (GI > am))
        cand = jnp.where(keep, G, inf)
        m = jnp.min(cand, axis=1, keepdims=True)
        am = jnp.min(jnp.where(cand == m, GI, big), axis=1, keepdims=True)
        p = lax.bitwise_and(am, Q - 1)
        hit = iq == p
        for j in range(g):
            vj = jnp.min(jnp.where(hit, D[j], inf), axis=1, keepdims=True)
            cv = jnp.where(selc == s * g + j, vj, cv)
            ci = jnp.where(selc == s * g + j, j * Q + p, ci)
        return m, am, cv, ci

    init = (jnp.full((R, 1), -inf, jnp.float32),
            jnp.full((R, 1), -1, jnp.int32),
            jnp.zeros((R, nc), jnp.float32),
            jnp.zeros((R, nc), jnp.int32))
    _, _, cv, ci = lax.fori_loop(0, nsel, step, init)
    return _lex_select(cv, ci, nsel, unroll=4 if nsel % 4 == 0 else nsel)


def _topk_body(N, nsel, x_ref, c_ref, o_ref):
    x = x_ref[0]
    c = c_ref[0]
    d = _sq_dist(c, x)
    Rb = d.shape[0]
    if N >= 2048:
        _, idx = _knn_select_grouped(d, nsel, 4)
    else:
        iota = lax.broadcasted_iota(jnp.int32, (Rb, N), 1)
        _, idx = _knn_select(d, iota, nsel)
    b = pl.program_id(0)
    o_ref[0] = idx + b * N


def _topk(xyzp, cent, nsel, rb):
    B, N, _ = xyzp.shape
    npoint = cent.shape[1]
    body = functools.partial(_topk_body, N, nsel)
    return pl.pallas_call(
        body,
        grid=(B, npoint // rb),
        in_specs=[
            pl.BlockSpec((1, N, 16), lambda b, r: (b, 0, 0)),
            pl.BlockSpec((1, rb, 16), lambda b, r: (b, r, 0)),
        ],
        out_specs=pl.BlockSpec((1, rb, nsel), lambda b, r: (b, r, 0)),
        out_shape=jax.ShapeDtypeStruct((B, npoint, nsel), jnp.int32),
    )(xyzp, cent)


def _sc_gather(table, idx, chunk):
    V, Cp = table.shape
    (btot,) = idx.shape
    info = plsc.get_sparse_core_info()
    nc, ns = info.num_cores, info.num_subcores
    nw = nc * ns
    per_w = btot // nw
    assert per_w * nw == btot and per_w % chunk == 0 and chunk % 8 == 0
    nch = per_w // chunk
    mesh = plsc.VectorSubcoreMesh(core_axis_name="c", subcore_axis_name="s")

    @functools.partial(
        pl.kernel, mesh=mesh,
        out_type=jax.ShapeDtypeStruct((btot, Cp), jnp.float32),
        compiler_params=pltpu.CompilerParams(use_tc_tiling_on_sc=False),
        scratch_types=[
            pltpu.VMEM((chunk,), jnp.int32),
            pltpu.VMEM((chunk, Cp), jnp.float32),
            pltpu.SemaphoreType.DMA,
        ],
    )
    def k(table_hbm, idx_hbm, out_hbm, idx_v, rows_v, sem):
        wid = lax.axis_index("s") * nc + lax.axis_index("c")
        for j in range(nch):
            base = wid * per_w + j * chunk
            pltpu.sync_copy(idx_hbm.at[pl.ds(base, chunk)], idx_v)
            pltpu.async_copy(table_hbm.at[idx_v], rows_v, sem).wait()
            pltpu.sync_copy(rows_v, out_hbm.at[pl.ds(base, chunk)])

    return k(table, idx)


def _sa_mlp_body(nlayer, *refs):
    g_ref, c_ref = refs[0], refs[1]
    wrefs = refs[2:2 + 2 * nlayer]
    o_ref = refs[-1]
    g = g_ref[...]
    c = c_ref[...]
    bc, ns, cp = g.shape
    x = (g - c[:, None, :]).reshape(bc * ns, cp)
    for i in range(nlayer):
        w = wrefs[2 * i][...]
        b = wrefs[2 * i + 1][...]
        x = jax.nn.relu(
            lax.dot_general(x, w, (((1,), (0,)), ((), ())),
                            preferred_element_type=jnp.float32) + b)
    cout = x.shape[-1]
    o_ref[...] = jnp.max(x.reshape(bc, ns, cout), axis=1)


def _sa_mlp(g, centp, ws, bc):
    M, ns, cp = g.shape
    cout = ws[-1][0].shape[1]
    nlayer = len(ws)
    args = [g, centp]
    in_specs = [
        pl.BlockSpec((bc, ns, cp), lambda i: (i, 0, 0)),
        pl.BlockSpec((bc, cp), lambda i: (i, 0)),
    ]
    for (w, b) in ws:
        args += [w, b.reshape(1, -1)]
        in_specs += [
            pl.BlockSpec(w.shape, lambda i: (0, 0)),
            pl.BlockSpec((1, b.shape[0]), lambda i: (0, 0)),
        ]
    body = functools.partial(_sa_mlp_body, nlayer)
    return pl.pallas_call(
        body,
        grid=(M // bc,),
        in_specs=in_specs,
        out_specs=pl.BlockSpec((bc, cout), lambda i: (i, 0)),
        out_shape=jax.ShapeDtypeStruct((M, cout), jnp.float32),
    )(*args)


def _fp_body(nlayer, has_fc, N2, *refs):
    i = 0
    x1_ref = refs[i]; i += 1
    x2_ref = refs[i]; i += 1
    f1_ref = refs[i]; i += 1
    f2_ref = refs[i]; i += 1
    w1a_ref = refs[i]; i += 1
    w1b_ref = refs[i]; i += 1
    b1_ref = refs[i]; i += 1
    rest = refs[i:-1]
    o_ref = refs[-1]

    x1 = x1_ref[0]
    x2 = x2_ref[0]
    d = _sq_dist(x1, x2)
    Rb = d.shape[0]
    iota = lax.broadcasted_iota(jnp.int32, (Rb, N2), 1)
    vals, idxs = _knn_select(d, iota, 3)
    dist = jnp.maximum(vals, 1e-10)
    w = 1.0 / dist
    w = w / jnp.sum(w, axis=1, keepdims=True)
    wsp = jnp.zeros_like(d)
    for s in range(3):
        wsp = wsp + jnp.where(iota == idxs[:, s][:, None], w[:, s][:, None], 0.0)
    interp = lax.dot_general(wsp, f2_ref[0], (((1,), (0,)), ((), ())),
                             preferred_element_type=jnp.float32)
    h = (lax.dot_general(interp, w1b_ref[...], (((1,), (0,)), ((), ())),
                         preferred_element_type=jnp.float32)
         + lax.dot_general(f1_ref[0], w1a_ref[...], (((1,), (0,)), ((), ())),
                           preferred_element_type=jnp.float32)
         + b1_ref[...])
    x = jax.nn.relu(h)
    for l in range(nlayer - 1):
        w_ = rest[2 * l][...]
        b_ = rest[2 * l + 1][...]
        x = jax.nn.relu(
            lax.dot_general(x, w_, (((1,), (0,)), ((), ())),
                            preferred_element_type=jnp.float32) + b_)
    if has_fc:
        wf1 = rest[2 * (nlayer - 1)][...]
        bf1 = rest[2 * (nlayer - 1) + 1][...]
        wf2 = rest[2 * (nlayer - 1) + 2][...]
        bf2 = rest[2 * (nlayer - 1) + 3][...]
        x = jax.nn.relu(
            lax.dot_general(x, wf1, (((1,), (0,)), ((), ())),
                            preferred_element_type=jnp.float32) + bf1)
        x = (lax.dot_general(x, wf2, (((1,), (0,)), ((), ())),
                             preferred_element_type=jnp.float32) + bf2)
        x = x[:, :o_ref.shape[-1]]
    o_ref[0] = x


def _fp_stage(x1p, x2p, f1, f2, ws, rb, fc=None):
    B, N1, _ = x1p.shape
    N2 = x2p.shape[1]
    c1p = f1.shape[-1]
    w1, b1 = ws[0]
    c1 = w1.shape[0] - f2.shape[-1]
    w1a = _pad_rows(w1[:c1], c1p)
    w1b = w1[c1:]
    nlayer = len(ws)
    args = [x1p, x2p, f1, f2, w1a, w1b, b1.reshape(1, -1)]
    in_specs = [
        pl.BlockSpec((1, rb, 16), lambda b, r: (b, r, 0)),
        pl.BlockSpec((1, N2, 16), lambda b, r: (b, 0, 0)),
        pl.BlockSpec((1, rb, c1p), lambda b, r: (b, r, 0)),
        pl.BlockSpec((1, N2, f2.shape[-1]), lambda b, r: (b, 0, 0)),
        pl.BlockSpec(w1a.shape, lambda b, r: (0, 0)),
        pl.BlockSpec(w1b.shape, lambda b, r: (0, 0)),
        pl.BlockSpec((1, b1.shape[0]), lambda b, r: (0, 0)),
    ]
    for (w_, b_) in ws[1:]:
        args += [w_, b_.reshape(1, -1)]
        in_specs += [
            pl.BlockSpec(w_.shape, lambda b, r: (0, 0)),
            pl.BlockSpec((1, b_.shape[0]), lambda b, r: (0, 0)),
        ]
    if fc is not None:
        (wf1, bf1), (wf2, bf2) = fc
        wf2p = _pad_last(wf2, 16)
        bf2p = _pad_last(bf2.reshape(1, -1), 16)
        args += [wf1, bf1.reshape(1, -1), wf2p, bf2p]
        in_specs += [
            pl.BlockSpec(wf1.shape, lambda b, r: (0, 0)),
            pl.BlockSpec((1, bf1.shape[0]), lambda b, r: (0, 0)),
            pl.BlockSpec(wf2p.shape, lambda b, r: (0, 0)),
            pl.BlockSpec((1, 16), lambda b, r: (0, 0)),
        ]
        cout = wf2.shape[1]
    else:
        cout = ws[-1][0].shape[1]
    body = functools.partial(_fp_body, nlayer, fc is not None, N2)
    return pl.pallas_call(
        body,
        grid=(B, N1 // rb),
        in_specs=in_specs,
        out_specs=pl.BlockSpec((1, rb, cout), lambda b, r: (b, r, 0)),
        out_shape=jax.ShapeDtypeStruct((B, N1, cout), jnp.float32),
    )(*args)


_SA_CFG = [
    (1024, 1024, 512, 2048),
    (256, 256, 256, 1024),
    (64, 64, 64, 512),
    (16, 16, 16, 128),
]


def _sa_stage(xyzp, feats, ws, npoint, rb, bc, chunk):
    B, N, _ = xyzp.shape
    C = feats.shape[-1]
    cp = _ceil_to(3 + C, 16)
    stride = N // npoint
    cent = xyzp[:, ::stride]
    knn = _topk(xyzp, cent, _NSAMPLE, rb)
    xf = _pad_last(jnp.concatenate([xyzp[..., :3], feats], axis=-1), cp)
    g = _sc_gather(xf.reshape(B * N, cp), knn.reshape(-1), chunk)
    centp = _pad_last(cent[..., :3], cp).reshape(B * npoint, cp)
    wpad = [(_pad_rows(ws[0][0], cp), ws[0][1])] + list(ws[1:])
    nf = _sa_mlp(g.reshape(B * npoint, _NSAMPLE, cp), centp, wpad, bc)
    return cent, nf.reshape(B, npoint, -1)


def kernel(pointcloud, params):
    B, N, _ = pointcloud.shape
    xyzp = _pad_last(pointcloud[..., :3], 16)
    feats0 = pointcloud[..., 3:]

    l_xyz = [xyzp]
    l_f = [feats0]
    for i, (npoint, rb, bc, chunk) in enumerate(_SA_CFG):
        cent, nf = _sa_stage(l_xyz[i], l_f[i], params["sa"][i], npoint, rb, bc, chunk)
        l_xyz.append(cent)
        l_f.append(nf)

    l_f[3] = _fp_stage(l_xyz[3], l_xyz[4], l_f[3], l_f[4], params["fp"][3], rb=64)
    l_f[2] = _fp_stage(l_xyz[2], l_xyz[3], l_f[2], l_f[3], params["fp"][2], rb=256)
    l_f[1] = _fp_stage(l_xyz[1], l_xyz[2], l_f[1], l_f[2], params["fp"][1], rb=1024)
    out = _fp_stage(l_xyz[0], l_xyz[1], l_f[0], l_f[1],
                    params["fp"][0], rb=2048, fc=params["fc"])
    return out


# speedup vs baseline: 1.0761x; 1.0232x over previous
import functools

import jax
import jax.numpy as jnp
from jax import lax
from jax.experimental import pallas as pl
from jax.experimental.pallas import tpu as pltpu
from jax.experimental.pallas import tpu_sc as plsc

_NSAMPLE = 32


def _ceil_to(x, m):
    return (x + m - 1) // m * m


def _pad_last(x, to):
    pad = to - x.shape[-1]
    if pad == 0:
        return x
    return jnp.pad(x, [(0, 0)] * (x.ndim - 1) + [(0, pad)])


def _pad_rows(w, to):
    pad = to - w.shape[0]
    if pad == 0:
        return w
    return jnp.pad(w, [(0, pad), (0, 0)])


def _knn_select(d, iota, nsel):
    R, N = d.shape
    inf = jnp.float32(jnp.inf)
    sel = lax.broadcasted_iota(jnp.int32, (R, nsel), 1)
    if R * N <= 64 * 256:
        unroll = nsel
    else:
        unroll = 4 if nsel % 4 == 0 else nsel

    def step(s, carry):
        m, am, out_v, out_i = carry
        for k in range(unroll):
            keep = (d > m) | ((d == m) & (iota > am))
            cand = jnp.where(keep, d, inf)
            m = jnp.min(cand, axis=1, keepdims=True)
            am = jnp.min(jnp.where(cand == m, iota, N), axis=1, keepdims=True)
            j = s * unroll + k
            out_v = jnp.where(sel == j, m, out_v)
            out_i = jnp.where(sel == j, am, out_i)
        return m, am, out_v, out_i

    init = (jnp.full((R, 1), -inf, jnp.float32),
            jnp.full((R, 1), -1, jnp.int32),
            jnp.zeros((R, nsel), jnp.float32),
            jnp.zeros((R, nsel), jnp.int32))
    _, am_, out_v, out_i = lax.fori_loop(0, nsel // unroll, step, init)
    return out_v, out_i


def _sq_dist(a, b):
    asq = jnp.sum(a * a, axis=1, keepdims=True)
    bsq = jnp.sum(b * b, axis=1)
    prod = lax.dot_general(a, b, (((1,), (1,)), ((), ())),
                           preferred_element_type=jnp.float32)
    return asq - 2.0 * prod + bsq[None, :]


def _lex_select(v, key, nsel, unroll):
    R, N = v.shape
    inf = jnp.float32(jnp.inf)
    big = jnp.int32(2**31 - 1)
    sel = lax.broadcasted_iota(jnp.int32, (R, nsel), 1)

    def step(s, carry):
        m, am, out_v, out_i = carry
        for k in range(unroll):
            keep = (v > m) | ((v == m) & (key > am))
            cand = jnp.where(keep, v, inf)
            m = jnp.min(cand, axis=1, keepdims=True)
            am = jnp.min(jnp.where(cand == m, key, big), axis=1, keepdims=True)
            j = s * unroll + k
            out_v = jnp.where(sel == j, m, out_v)
            out_i = jnp.where(sel == j, am, out_i)
        return m, am, out_v, out_i

    init = (jnp.full((R, 1), -inf, jnp.float32),
            jnp.full((R, 1), -1, jnp.int32),
            jnp.zeros((R, nsel), jnp.float32),
            jnp.zeros((R, nsel), jnp.int32))
    _, _, out_v, out_i = lax.fori_loop(0, nsel // unroll, step, init)
    return out_v, out_i


def _knn_select_grouped(d, nsel, g):
    R, N = d.shape
    Q = N // g
    D = [d[:, j * Q:(j + 1) * Q] for j in range(g)]
    iq = lax.broadcasted_iota(jnp.int32, (R, Q), 1)
    G = D[0]
    GI = iq
    for j in range(1, g):
        lt = D[j] < G
        G = jnp.where(lt, D[j], G)
        GI = jnp.where(lt, j * Q + iq, GI)

    inf = jnp.float32(jnp.inf)
    big = jnp.int32(2**31 - 1)
    nc = g * nsel
    selc = lax.broadcasted_iota(jnp.int32, (R, nc), 1)

    un1 = 2 if nsel % 2 == 0 else 1

    def step(s, carry):
        m, am, cv, ci = carry
        for k in range(un1):
            keep = (G > m) | ((G == m) & (GI > am))
            cand = jnp.where(keep, G, inf)
            m = jnp.min(cand, axis=1, keepdims=True)
            am = jnp.min(jnp.where(cand == m, GI, big), axis=1, keepdims=True)
            p = lax.bitwise_and(am, Q - 1)
            hit = iq == p
            for j in range(g):
                vj = jnp.min(jnp.where(hit, D[j], inf), axis=1, keepdims=True)
                sl = (s * un1 + k) * g + j
                cv = jnp.where(selc == sl, vj, cv)
                ci = jnp.where(selc == sl, j * Q + p, ci)
        return m, am, cv, ci

    init = (jnp.full((R, 1), -inf, jnp.float32),
            jnp.full((R, 1), -1, jnp.int32),
            jnp.zeros((R, nc), jnp.float32),
            jnp.zeros((R, nc), jnp.int32))
    _, _, cv, ci = lax.fori_loop(0, nsel // un1, step, init)
    return _lex_select(cv, ci, nsel, unroll=4 if nsel % 4 == 0 else nsel)


def _topk_body(N, nsel, x_ref, c_ref, o_ref):
    x = x_ref[0]
    c = c_ref[0]
    d = _sq_dist(c, x)
    Rb = d.shape[0]
    if N >= 2048:
        _, idx = _knn_select_grouped(d, nsel, 4)
    else:
        iota = lax.broadcasted_iota(jnp.int32, (Rb, N), 1)
        _, idx = _knn_select(d, iota, nsel)
    b = pl.program_id(0)
    o_ref[0] = idx + b * N


def _topk(xyzp, cent, nsel, rb):
    B, N, _ = xyzp.shape
    npoint = cent.shape[1]
    body = functools.partial(_topk_body, N, nsel)
    return pl.pallas_call(
        body,
        grid=(B, npoint // rb),
        in_specs=[
            pl.BlockSpec((1, N, 16), lambda b, r: (b, 0, 0)),
            pl.BlockSpec((1, rb, 16), lambda b, r: (b, r, 0)),
        ],
        out_specs=pl.BlockSpec((1, rb, nsel), lambda b, r: (b, r, 0)),
        out_shape=jax.ShapeDtypeStruct((B, npoint, nsel), jnp.int32),
    )(xyzp, cent)


def _sc_gather(table, idx, chunk):
    V, Cp = table.shape
    (btot,) = idx.shape
    info = plsc.get_sparse_core_info()
    nc, ns = info.num_cores, info.num_subcores
    nw = nc * ns
    per_w = btot // nw
    assert per_w * nw == btot and per_w % chunk == 0 and chunk % 8 == 0
    nch = per_w // chunk
    mesh = plsc.VectorSubcoreMesh(core_axis_name="c", subcore_axis_name="s")

    @functools.partial(
        pl.kernel, mesh=mesh,
        out_type=jax.ShapeDtypeStruct((btot, Cp), jnp.float32),
        compiler_params=pltpu.CompilerParams(use_tc_tiling_on_sc=False),
        scratch_types=[
            pltpu.VMEM((chunk,), jnp.int32),
            pltpu.VMEM((chunk, Cp), jnp.float32),
            pltpu.SemaphoreType.DMA,
        ],
    )
    def k(table_hbm, idx_hbm, out_hbm, idx_v, rows_v, sem):
        wid = lax.axis_index("s") * nc + lax.axis_index("c")
        for j in range(nch):
            base = wid * per_w + j * chunk
            pltpu.sync_copy(idx_hbm.at[pl.ds(base, chunk)], idx_v)
            pltpu.async_copy(table_hbm.at[idx_v], rows_v, sem).wait()
            pltpu.sync_copy(rows_v, out_hbm.at[pl.ds(base, chunk)])

    return k(table, idx)


def _sa_mlp_body(nlayer, *refs):
    g_ref, c_ref = refs[0], refs[1]
    wrefs = refs[2:2 + 2 * nlayer]
    o_ref = refs[-1]
    g = g_ref[...]
    c = c_ref[...]
    bc, ns, cp = g.shape
    x = (g - c[:, None, :]).reshape(bc * ns, cp)
    for i in range(nlayer):
        w = wrefs[2 * i][...]
        b = wrefs[2 * i + 1][...]
        x = jax.nn.relu(
            lax.dot_general(x, w, (((1,), (0,)), ((), ())),
                            preferred_element_type=jnp.float32) + b)
    cout = x.shape[-1]
    o_ref[...] = jnp.max(x.reshape(bc, ns, cout), axis=1)


def _sa_mlp(g, centp, ws, bc):
    M, ns, cp = g.shape
    cout = ws[-1][0].shape[1]
    nlayer = len(ws)
    args = [g, centp]
    in_specs = [
        pl.BlockSpec((bc, ns, cp), lambda i: (i, 0, 0)),
        pl.BlockSpec((bc, cp), lambda i: (i, 0)),
    ]
    for (w, b) in ws:
        args += [w, b.reshape(1, -1)]
        in_specs += [
            pl.BlockSpec(w.shape, lambda i: (0, 0)),
            pl.BlockSpec((1, b.shape[0]), lambda i: (0, 0)),
        ]
    body = functools.partial(_sa_mlp_body, nlayer)
    return pl.pallas_call(
        body,
        grid=(M // bc,),
        in_specs=in_specs,
        out_specs=pl.BlockSpec((bc, cout), lambda i: (i, 0)),
        out_shape=jax.ShapeDtypeStruct((M, cout), jnp.float32),
    )(*args)


def _fp_body(nlayer, has_fc, N2, *refs):
    i = 0
    x1_ref = refs[i]; i += 1
    x2_ref = refs[i]; i += 1
    f1_ref = refs[i]; i += 1
    f2_ref = refs[i]; i += 1
    w1a_ref = refs[i]; i += 1
    w1b_ref = refs[i]; i += 1
    b1_ref = refs[i]; i += 1
    rest = refs[i:-1]
    o_ref = refs[-1]

    x1 = x1_ref[0]
    x2 = x2_ref[0]
    d = _sq_dist(x1, x2)
    Rb = d.shape[0]
    iota = lax.broadcasted_iota(jnp.int32, (Rb, N2), 1)
    vals, idxs = _knn_select(d, iota, 3)
    dist = jnp.maximum(vals, 1e-10)
    w = 1.0 / dist
    w = w / jnp.sum(w, axis=1, keepdims=True)
    wsp = jnp.zeros_like(d)
    for s in range(3):
        wsp = wsp + jnp.where(iota == idxs[:, s][:, None], w[:, s][:, None], 0.0)
    interp = lax.dot_general(wsp, f2_ref[0], (((1,), (0,)), ((), ())),
                             preferred_element_type=jnp.float32)
    h = (lax.dot_general(interp, w1b_ref[...], (((1,), (0,)), ((), ())),
                         preferred_element_type=jnp.float32)
         + lax.dot_general(f1_ref[0], w1a_ref[...], (((1,), (0,)), ((), ())),
                           preferred_element_type=jnp.float32)
         + b1_ref[...])
    x = jax.nn.relu(h)
    for l in range(nlayer - 1):
        w_ = rest[2 * l][...]
        b_ = rest[2 * l + 1][...]
        x = jax.nn.relu(
            lax.dot_general(x, w_, (((1,), (0,)), ((), ())),
                            preferred_element_type=jnp.float32) + b_)
    if has_fc:
        wf1 = rest[2 * (nlayer - 1)][...]
        bf1 = rest[2 * (nlayer - 1) + 1][...]
        wf2 = rest[2 * (nlayer - 1) + 2][...]
        bf2 = rest[2 * (nlayer - 1) + 3][...]
        x = jax.nn.relu(
            lax.dot_general(x, wf1, (((1,), (0,)), ((), ())),
                            preferred_element_type=jnp.float32) + bf1)
        x = (lax.dot_general(x, wf2, (((1,), (0,)), ((), ())),
                             preferred_element_type=jnp.float32) + bf2)
        x = x[:, :o_ref.shape[-1]]
    o_ref[0] = x


def _fp_stage(x1p, x2p, f1, f2, ws, rb, fc=None):
    B, N1, _ = x1p.shape
    N2 = x2p.shape[1]
    c1p = f1.shape[-1]
    w1, b1 = ws[0]
    c1 = w1.shape[0] - f2.shape[-1]
    w1a = _pad_rows(w1[:c1], c1p)
    w1b = w1[c1:]
    nlayer = len(ws)
    args = [x1p, x2p, f1, f2, w1a, w1b, b1.reshape(1, -1)]
    in_specs = [
        pl.BlockSpec((1, rb, 16), lambda b, r: (b, r, 0)),
        pl.BlockSpec((1, N2, 16), lambda b, r: (b, 0, 0)),
        pl.BlockSpec((1, rb, c1p), lambda b, r: (b, r, 0)),
        pl.BlockSpec((1, N2, f2.shape[-1]), lambda b, r: (b, 0, 0)),
        pl.BlockSpec(w1a.shape, lambda b, r: (0, 0)),
        pl.BlockSpec(w1b.shape, lambda b, r: (0, 0)),
        pl.BlockSpec((1, b1.shape[0]), lambda b, r: (0, 0)),
    ]
    for (w_, b_) in ws[1:]:
        args += [w_, b_.reshape(1, -1)]
        in_specs += [
            pl.BlockSpec(w_.shape, lambda b, r: (0, 0)),
            pl.BlockSpec((1, b_.shape[0]), lambda b, r: (0, 0)),
        ]
    if fc is not None:
        (wf1, bf1), (wf2, bf2) = fc
        wf2p = _pad_last(wf2, 16)
        bf2p = _pad_last(bf2.reshape(1, -1), 16)
        args += [wf1, bf1.reshape(1, -1), wf2p, bf2p]
        in_specs += [
            pl.BlockSpec(wf1.shape, lambda b, r: (0, 0)),
            pl.BlockSpec((1, bf1.shape[0]), lambda b, r: (0, 0)),
            pl.BlockSpec(wf2p.shape, lambda b, r: (0, 0)),
            pl.BlockSpec((1, 16), lambda b, r: (0, 0)),
        ]
        cout = wf2.shape[1]
    else:
        cout = ws[-1][0].shape[1]
    body = functools.partial(_fp_body, nlayer, fc is not None, N2)
    return pl.pallas_call(
        body,
        grid=(B, N1 // rb),
        in_specs=in_specs,
        out_specs=pl.BlockSpec((1, rb, cout), lambda b, r: (b, r, 0)),
        out_shape=jax.ShapeDtypeStruct((B, N1, cout), jnp.float32),
    )(*args)


_SA_CFG = [
    (1024, 1024, 1024, 2048),
    (256, 256, 256, 1024),
    (64, 64, 64, 512),
    (16, 16, 16, 128),
]


def _sa_stage(xyzp, feats, ws, npoint, rb, bc, chunk):
    B, N, _ = xyzp.shape
    C = feats.shape[-1]
    cp = _ceil_to(3 + C, 16)
    stride = N // npoint
    cent = xyzp[:, ::stride]
    knn = _topk(xyzp, cent, _NSAMPLE, rb)
    xf = _pad_last(jnp.concatenate([xyzp[..., :3], feats], axis=-1), cp)
    g = _sc_gather(xf.reshape(B * N, cp), knn.reshape(-1), chunk)
    centp = _pad_last(cent[..., :3], cp).reshape(B * npoint, cp)
    wpad = [(_pad_rows(ws[0][0], cp), ws[0][1])] + list(ws[1:])
    nf = _sa_mlp(g.reshape(B * npoint, _NSAMPLE, cp), centp, wpad, bc)
    return cent, nf.reshape(B, npoint, -1)


def kernel(pointcloud, params):
    B, N, _ = pointcloud.shape
    xyzp = _pad_last(pointcloud[..., :3], 16)
    feats0 = pointcloud[..., 3:]

    l_xyz = [xyzp]
    l_f = [feats0]
    for i, (npoint, rb, bc, chunk) in enumerate(_SA_CFG):
        cent, nf = _sa_stage(l_xyz[i], l_f[i], params["sa"][i], npoint, rb, bc, chunk)
        l_xyz.append(cent)
        l_f.append(nf)

    l_f[3] = _fp_stage(l_xyz[3], l_xyz[4], l_f[3], l_f[4], params["fp"][3], rb=64)
    l_f[2] = _fp_stage(l_xyz[2], l_xyz[3], l_f[2], l_f[3], params["fp"][2], rb=256)
    l_f[1] = _fp_stage(l_xyz[1], l_xyz[2], l_f[1], l_f[2], params["fp"][1], rb=1024)
    out = _fp_stage(l_xyz[0], l_xyz[1], l_f[0], l_f[1],
                    params["fp"][0], rb=2048, fc=params["fc"])
    return out


# speedup vs baseline: 1.0864x; 1.0096x over previous
import functools

import jax
import jax.numpy as jnp
from jax import lax
from jax.experimental import pallas as pl
from jax.experimental.pallas import tpu as pltpu
from jax.experimental.pallas import tpu_sc as plsc

_NSAMPLE = 32


def _ceil_to(x, m):
    return (x + m - 1) // m * m


def _pad_last(x, to):
    pad = to - x.shape[-1]
    if pad == 0:
        return x
    return jnp.pad(x, [(0, 0)] * (x.ndim - 1) + [(0, pad)])


def _pad_rows(w, to):
    pad = to - w.shape[0]
    if pad == 0:
        return w
    return jnp.pad(w, [(0, pad), (0, 0)])


def _knn_select(d, iota, nsel):
    R, N = d.shape
    inf = jnp.float32(jnp.inf)
    sel = lax.broadcasted_iota(jnp.int32, (R, nsel), 1)
    if R * N <= 64 * 256:
        unroll = nsel
    else:
        unroll = 4 if nsel % 4 == 0 else nsel

    def step(s, carry):
        m, am, out_v, out_i = carry
        for k in range(unroll):
            keep = (d > m) | ((d == m) & (iota > am))
            cand = jnp.where(keep, d, inf)
            m = jnp.min(cand, axis=1, keepdims=True)
            am = jnp.min(jnp.where(cand == m, iota, N), axis=1, keepdims=True)
            j = s * unroll + k
            out_v = jnp.where(sel == j, m, out_v)
            out_i = jnp.where(sel == j, am, out_i)
        return m, am, out_v, out_i

    init = (jnp.full((R, 1), -inf, jnp.float32),
            jnp.full((R, 1), -1, jnp.int32),
            jnp.zeros((R, nsel), jnp.float32),
            jnp.zeros((R, nsel), jnp.int32))
    _, am_, out_v, out_i = lax.fori_loop(0, nsel // unroll, step, init)
    return out_v, out_i


def _sq_dist(a, b):
    asq = jnp.sum(a * a, axis=1, keepdims=True)
    bsq = jnp.sum(b * b, axis=1)
    prod = lax.dot_general(a, b, (((1,), (1,)), ((), ())),
                           preferred_element_type=jnp.float32)
    return asq - 2.0 * prod + bsq[None, :]


def _lex_select(v, key, nsel, unroll):
    R, N = v.shape
    inf = jnp.float32(jnp.inf)
    big = jnp.int32(2**31 - 1)
    sel = lax.broadcasted_iota(jnp.int32, (R, nsel), 1)

    def step(s, carry):
        m, am, out_v, out_i = carry
        for k in range(unroll):
            keep = (v > m) | ((v == m) & (key > am))
            cand = jnp.where(keep, v, inf)
            m = jnp.min(cand, axis=1, keepdims=True)
            am = jnp.min(jnp.where(cand == m, key, big), axis=1, keepdims=True)
            j = s * unroll + k
            out_v = jnp.where(sel == j, m, out_v)
            out_i = jnp.where(sel == j, am, out_i)
        return m, am, out_v, out_i

    init = (jnp.full((R, 1), -inf, jnp.float32),
            jnp.full((R, 1), -1, jnp.int32),
            jnp.zeros((R, nsel), jnp.float32),
            jnp.zeros((R, nsel), jnp.int32))
    _, _, out_v, out_i = lax.fori_loop(0, nsel // unroll, step, init)
    return out_v, out_i


def _knn_select_grouped(d, nsel, g):
    R, N = d.shape
    Q = N // g
    D = [d[:, j * Q:(j + 1) * Q] for j in range(g)]
    iq = lax.broadcasted_iota(jnp.int32, (R, Q), 1)
    G = D[0]
    GI = iq
    for j in range(1, g):
        lt = D[j] < G
        G = jnp.where(lt, D[j], G)
        GI = jnp.where(lt, j * Q + iq, GI)

    inf = jnp.float32(jnp.inf)
    big = jnp.int32(2**31 - 1)
    nc = g * nsel
    selc = lax.broadcasted_iota(jnp.int32, (R, nc), 1)

    un1 = 4 if nsel % 4 == 0 else 1

    def step(s, carry):
        m, am, cv, ci = carry
        for k in range(un1):
            keep = (G > m) | ((G == m) & (GI > am))
            cand = jnp.where(keep, G, inf)
            m = jnp.min(cand, axis=1, keepdims=True)
            am = jnp.min(jnp.where(cand == m, GI, big), axis=1, keepdims=True)
            p = lax.bitwise_and(am, Q - 1)
            hit = iq == p
            for j in range(g):
                vj = jnp.min(jnp.where(hit, D[j], inf), axis=1, keepdims=True)
                sl = (s * un1 + k) * g + j
                cv = jnp.where(selc == sl, vj, cv)
                ci = jnp.where(selc == sl, j * Q + p, ci)
        return m, am, cv, ci

    init = (jnp.full((R, 1), -inf, jnp.float32),
            jnp.full((R, 1), -1, jnp.int32),
            jnp.zeros((R, nc), jnp.float32),
            jnp.zeros((R, nc), jnp.int32))
    _, _, cv, ci = lax.fori_loop(0, nsel // un1, step, init)
    return _lex_select(cv, ci, nsel, unroll=4 if nsel % 4 == 0 else nsel)


def _topk_body(N, nsel, x_ref, c_ref, o_ref):
    x = x_ref[0]
    c = c_ref[0]
    d = _sq_dist(c, x)
    Rb = d.shape[0]
    if N >= 2048:
        _, idx = _knn_select_grouped(d, nsel, 4)
    else:
        iota = lax.broadcasted_iota(jnp.int32, (Rb, N), 1)
        _, idx = _knn_select(d, iota, nsel)
    b = pl.program_id(0)
    o_ref[0] = idx + b * N


def _topk(xyzp, cent, nsel, rb):
    B, N, _ = xyzp.shape
    npoint = cent.shape[1]
    body = functools.partial(_topk_body, N, nsel)
    return pl.pallas_call(
        body,
        grid=(B, npoint // rb),
        in_specs=[
            pl.BlockSpec((1, N, 16), lambda b, r: (b, 0, 0)),
            pl.BlockSpec((1, rb, 16), lambda b, r: (b, r, 0)),
        ],
        out_specs=pl.BlockSpec((1, rb, nsel), lambda b, r: (b, r, 0)),
        out_shape=jax.ShapeDtypeStruct((B, npoint, nsel), jnp.int32),
    )(xyzp, cent)


def _sc_gather(table, idx, chunk):
    V, Cp = table.shape
    (btot,) = idx.shape
    info = plsc.get_sparse_core_info()
    nc, ns = info.num_cores, info.num_subcores
    nw = nc * ns
    per_w = btot // nw
    assert per_w * nw == btot and per_w % chunk == 0 and chunk % 8 == 0
    nch = per_w // chunk
    mesh = plsc.VectorSubcoreMesh(core_axis_name="c", subcore_axis_name="s")

    @functools.partial(
        pl.kernel, mesh=mesh,
        out_type=jax.ShapeDtypeStruct((btot, Cp), jnp.float32),
        compiler_params=pltpu.CompilerParams(use_tc_tiling_on_sc=False),
        scratch_types=[
            pltpu.VMEM((chunk,), jnp.int32),
            pltpu.VMEM((chunk, Cp), jnp.float32),
            pltpu.SemaphoreType.DMA,
        ],
    )
    def k(table_hbm, idx_hbm, out_hbm, idx_v, rows_v, sem):
        wid = lax.axis_index("s") * nc + lax.axis_index("c")
        for j in range(nch):
            base = wid * per_w + j * chunk
            pltpu.sync_copy(idx_hbm.at[pl.ds(base, chunk)], idx_v)
            pltpu.async_copy(table_hbm.at[idx_v], rows_v, sem).wait()
            pltpu.sync_copy(rows_v, out_hbm.at[pl.ds(base, chunk)])

    return k(table, idx)


def _sa_mlp_body(nlayer, *refs):
    g_ref, c_ref = refs[0], refs[1]
    wrefs = refs[2:2 + 2 * nlayer]
    o_ref = refs[-1]
    g = g_ref[...]
    c = c_ref[...]
    bc, ns, cp = g.shape
    x = (g - c[:, None, :]).reshape(bc * ns, cp)
    for i in range(nlayer):
        w = wrefs[2 * i][...]
        b = wrefs[2 * i + 1][...]
        x = jax.nn.relu(
            lax.dot_general(x, w, (((1,), (0,)), ((), ())),
                            preferred_element_type=jnp.float32) + b)
    cout = x.shape[-1]
    o_ref[...] = jnp.max(x.reshape(bc, ns, cout), axis=1)


def _sa_mlp(g, centp, ws, bc):
    M, ns, cp = g.shape
    cout = ws[-1][0].shape[1]
    nlayer = len(ws)
    args = [g, centp]
    in_specs = [
        pl.BlockSpec((bc, ns, cp), lambda i: (i, 0, 0)),
        pl.BlockSpec((bc, cp), lambda i: (i, 0)),
    ]
    for (w, b) in ws:
        args += [w, b.reshape(1, -1)]
        in_specs += [
            pl.BlockSpec(w.shape, lambda i: (0, 0)),
            pl.BlockSpec((1, b.shape[0]), lambda i: (0, 0)),
        ]
    body = functools.partial(_sa_mlp_body, nlayer)
    return pl.pallas_call(
        body,
        grid=(M // bc,),
        in_specs=in_specs,
        out_specs=pl.BlockSpec((bc, cout), lambda i: (i, 0)),
        out_shape=jax.ShapeDtypeStruct((M, cout), jnp.float32),
    )(*args)


def _fp_body(nlayer, has_fc, N2, *refs):
    i = 0
    x1_ref = refs[i]; i += 1
    x2_ref = refs[i]; i += 1
    f1_ref = refs[i]; i += 1
    f2_ref = refs[i]; i += 1
    w1a_ref = refs[i]; i += 1
    w1b_ref = refs[i]; i += 1
    b1_ref = refs[i]; i += 1
    rest = refs[i:-1]
    o_ref = refs[-1]

    x1 = x1_ref[0]
    x2 = x2_ref[0]
    d = _sq_dist(x1, x2)
    Rb = d.shape[0]
    iota = lax.broadcasted_iota(jnp.int32, (Rb, N2), 1)
    vals, idxs = _knn_select(d, iota, 3)
    dist = jnp.maximum(vals, 1e-10)
    w = 1.0 / dist
    w = w / jnp.sum(w, axis=1, keepdims=True)
    wsp = jnp.zeros_like(d)
    for s in range(3):
        wsp = wsp + jnp.where(iota == idxs[:, s][:, None], w[:, s][:, None], 0.0)
    interp = lax.dot_general(wsp, f2_ref[0], (((1,), (0,)), ((), ())),
                             preferred_element_type=jnp.float32)
    h = (lax.dot_general(interp, w1b_ref[...], (((1,), (0,)), ((), ())),
                         preferred_element_type=jnp.float32)
         + lax.dot_general(f1_ref[0], w1a_ref[...], (((1,), (0,)), ((), ())),
                           preferred_element_type=jnp.float32)
         + b1_ref[...])
    x = jax.nn.relu(h)
    for l in range(nlayer - 1):
        w_ = rest[2 * l][...]
        b_ = rest[2 * l + 1][...]
        x = jax.nn.relu(
            lax.dot_general(x, w_, (((1,), (0,)), ((), ())),
                            preferred_element_type=jnp.float32) + b_)
    if has_fc:
        wf1 = rest[2 * (nlayer - 1)][...]
        bf1 = rest[2 * (nlayer - 1) + 1][...]
        wf2 = rest[2 * (nlayer - 1) + 2][...]
        bf2 = rest[2 * (nlayer - 1) + 3][...]
        x = jax.nn.relu(
            lax.dot_general(x, wf1, (((1,), (0,)), ((), ())),
                            preferred_element_type=jnp.float32) + bf1)
        x = (lax.dot_general(x, wf2, (((1,), (0,)), ((), ())),
                             preferred_element_type=jnp.float32) + bf2)
        x = x[:, :o_ref.shape[-1]]
    o_ref[0] = x


def _fp_stage(x1p, x2p, f1, f2, ws, rb, fc=None):
    B, N1, _ = x1p.shape
    N2 = x2p.shape[1]
    c1p = f1.shape[-1]
    w1, b1 = ws[0]
    c1 = w1.shape[0] - f2.shape[-1]
    w1a = _pad_rows(w1[:c1], c1p)
    w1b = w1[c1:]
    nlayer = len(ws)
    args = [x1p, x2p, f1, f2, w1a, w1b, b1.reshape(1, -1)]
    in_specs = [
        pl.BlockSpec((1, rb, 16), lambda b, r: (b, r, 0)),
        pl.BlockSpec((1, N2, 16), lambda b, r: (b, 0, 0)),
        pl.BlockSpec((1, rb, c1p), lambda b, r: (b, r, 0)),
        pl.BlockSpec((1, N2, f2.shape[-1]), lambda b, r: (b, 0, 0)),
        pl.BlockSpec(w1a.shape, lambda b, r: (0, 0)),
        pl.BlockSpec(w1b.shape, lambda b, r: (0, 0)),
        pl.BlockSpec((1, b1.shape[0]), lambda b, r: (0, 0)),
    ]
    for (w_, b_) in ws[1:]:
        args += [w_, b_.reshape(1, -1)]
        in_specs += [
            pl.BlockSpec(w_.shape, lambda b, r: (0, 0)),
            pl.BlockSpec((1, b_.shape[0]), lambda b, r: (0, 0)),
        ]
    if fc is not None:
        (wf1, bf1), (wf2, bf2) = fc
        wf2p = _pad_last(wf2, 16)
        bf2p = _pad_last(bf2.reshape(1, -1), 16)
        args += [wf1, bf1.reshape(1, -1), wf2p, bf2p]
        in_specs += [
            pl.BlockSpec(wf1.shape, lambda b, r: (0, 0)),
            pl.BlockSpec((1, bf1.shape[0]), lambda b, r: (0, 0)),
            pl.BlockSpec(wf2p.shape, lambda b, r: (0, 0)),
            pl.BlockSpec((1, 16), lambda b, r: (0, 0)),
        ]
        cout = wf2.shape[1]
    else:
        cout = ws[-1][0].shape[1]
    body = functools.partial(_fp_body, nlayer, fc is not None, N2)
    return pl.pallas_call(
        body,
        grid=(B, N1 // rb),
        in_specs=in_specs,
        out_specs=pl.BlockSpec((1, rb, cout), lambda b, r: (b, r, 0)),
        out_shape=jax.ShapeDtypeStruct((B, N1, cout), jnp.float32),
    )(*args)


_SA_CFG = [
    (1024, 1024, 1024, 2048),
    (256, 256, 256, 1024),
    (64, 64, 64, 512),
    (16, 16, 16, 128),
]


def _sa_stage(xyzp, feats, ws, npoint, rb, bc, chunk):
    B, N, _ = xyzp.shape
    C = feats.shape[-1]
    cp = _ceil_to(3 + C, 16)
    stride = N // npoint
    cent = xyzp[:, ::stride]
    knn = _topk(xyzp, cent, _NSAMPLE, rb)
    xf = _pad_last(jnp.concatenate([xyzp[..., :3], feats], axis=-1), cp)
    g = _sc_gather(xf.reshape(B * N, cp), knn.reshape(-1), chunk)
    centp = _pad_last(cent[..., :3], cp).reshape(B * npoint, cp)
    wpad = [(_pad_rows(ws[0][0], cp), ws[0][1])] + list(ws[1:])
    nf = _sa_mlp(g.reshape(B * npoint, _NSAMPLE, cp), centp, wpad, bc)
    return cent, nf.reshape(B, npoint, -1)


def kernel(pointcloud, params):
    B, N, _ = pointcloud.shape
    xyzp = _pad_last(pointcloud[..., :3], 16)
    feats0 = pointcloud[..., 3:]

    l_xyz = [xyzp]
    l_f = [feats0]
    for i, (npoint, rb, bc, chunk) in enumerate(_SA_CFG):
        cent, nf = _sa_stage(l_xyz[i], l_f[i], params["sa"][i], npoint, rb, bc, chunk)
        l_xyz.append(cent)
        l_f.append(nf)

    l_f[3] = _fp_stage(l_xyz[3], l_xyz[4], l_f[3], l_f[4], params["fp"][3], rb=64)
    l_f[2] = _fp_stage(l_xyz[2], l_xyz[3], l_f[2], l_f[3], params["fp"][2], rb=256)
    l_f[1] = _fp_stage(l_xyz[1], l_xyz[2], l_f[1], l_f[2], params["fp"][1], rb=1024)
    out = _fp_stage(l_xyz[0], l_xyz[1], l_f[0], l_f[1],
                    params["fp"][0], rb=2048, fc=params["fc"])
    return out
